# baseline (speedup 1.0000x reference)
import functools

import jax
import jax.numpy as jnp
from jax import lax
from jax.experimental import pallas as pl
from jax.experimental.pallas import tpu as pltpu
from jax.experimental.pallas import tpu_sc as plsc

B = 16384
C = 10
NB = 16
RB = B // NB
NW = 32
RW = B // NW
FW = RW * C


_mesh = plsc.VectorSubcoreMesh(core_axis_name="c", subcore_axis_name="s")


@functools.partial(
    pl.kernel,
    out_type=jax.ShapeDtypeStruct((B * C,), jnp.float32),
    mesh=_mesh,
    scratch_types=[
        pltpu.VMEM((RW,), jnp.int32),
        pltpu.VMEM((FW,), jnp.float32),
    ],
    compiler_params=pltpu.CompilerParams(needs_layout_passes=False),
)
def _ohe_sc(gt_hbm, out_hbm, gbuf, obuf):
    wid = lax.axis_index("s") * 2 + lax.axis_index("c")
    base = wid * RW
    pltpu.sync_copy(gt_hbm.at[pl.ds(base, RW)], gbuf)

    zeros = jnp.zeros((16,), jnp.float32)

    def zstep(k, carry):
        for u in range(8):
            obuf[pl.ds(k * 128 + u * 16, 16)] = zeros
        return carry

    lax.fori_loop(0, FW // 128, zstep, 0)

    ones = zeros + 1.0

    def sstep(j, carry):
        for u in range(8):
            rows = lax.iota(jnp.int32, 16) + (j * 8 + u) * 16
            g = gbuf[pl.ds((j * 8 + u) * 16, 16)]
            plsc.store_scatter(obuf, [rows * C + g], ones)
        return carry

    lax.fori_loop(0, RW // 128, sstep, 0)
    pltpu.sync_copy(obuf, out_hbm.at[pl.ds(base * C, FW)])


def _softmax_body(x_hbm, soft_hbm, ebuf, xbuf, sbuf, xsem, ssem):
    def xcopy(b, slot):
        return pltpu.make_async_copy(
            x_hbm.at[pl.ds(b * RB, RB), :], xbuf.at[slot], xsem.at[slot])

    def scopy(b, slot):
        return pltpu.make_async_copy(
            sbuf.at[slot], soft_hbm.at[pl.ds(b * RB, RB), :], ssem.at[slot])

    xcopy(0, 0).start()
    xcopy(1, 1).start()

    def pass1(b, acc):
        slot = jax.lax.rem(b, 2)
        xcopy(b, slot).wait()
        e = jnp.exp(xbuf[slot])
        ebuf[pl.ds(b * RB, RB), :] = e

        @pl.when(b + 2 < NB)
        def _prefetch():
            xcopy(b + 2, slot).start()

        return acc + jnp.sum(e)

    total = jax.lax.fori_loop(0, NB, pass1, 0.0)
    inv = 1.0 / total

    def pass2(b, carry):
        slot = jax.lax.rem(b, 2)

        @pl.when(b >= 2)
        def _drain():
            scopy(b - 2, slot).wait()

        sbuf[slot] = ebuf[pl.ds(b * RB, RB), :] * inv
        scopy(b, slot).start()
        return carry

    jax.lax.fori_loop(0, NB, pass2, 0)
    scopy(NB - 2, 0).wait()
    scopy(NB - 1, 1).wait()


def _softmax_tc(x):
    return pl.pallas_call(
        _softmax_body,
        in_specs=[pl.BlockSpec(memory_space=pltpu.MemorySpace.HBM)],
        out_specs=pl.BlockSpec(memory_space=pltpu.MemorySpace.HBM),
        out_shape=jax.ShapeDtypeStruct((B, C), jnp.float32),
        scratch_shapes=[
            pltpu.VMEM((B, C), jnp.float32),
            pltpu.VMEM((2, RB, C), jnp.float32),
            pltpu.VMEM((2, RB, C), jnp.float32),
            pltpu.SemaphoreType.DMA((2,)),
            pltpu.SemaphoreType.DMA((2,)),
        ],
    )(x)


def kernel(inference, ground_truth):
    ohe_flat = _ohe_sc(ground_truth.astype(jnp.int32))
    soft = _softmax_tc(inference)
    return (soft, ohe_flat.reshape(B, C))


# speedup vs baseline: 1.1738x; 1.1738x over previous
import functools

import jax
import jax.numpy as jnp
from jax import lax
from jax.experimental import pallas as pl
from jax.experimental.pallas import tpu as pltpu
from jax.experimental.pallas import tpu_sc as plsc

B = 16384
C = 10
NB = 16
RB = B // NB
NW = 32
RW = B // NW
FW = RW * C


_mesh = plsc.VectorSubcoreMesh(core_axis_name="c", subcore_axis_name="s")


@functools.partial(
    pl.kernel,
    out_type=jax.ShapeDtypeStruct((B, 128), jnp.float32),
    mesh=_mesh,
    scratch_types=[
        pltpu.VMEM((RW,), jnp.int32),
        pltpu.VMEM((RW, 128), jnp.float32),
    ],
    compiler_params=pltpu.CompilerParams(needs_layout_passes=False),
)
def _ohe_sc(gt_hbm, out_hbm, gbuf, obuf):
    wid = lax.axis_index("s") * 2 + lax.axis_index("c")
    base = wid * RW
    pltpu.sync_copy(gt_hbm.at[pl.ds(base, RW)], gbuf)

    zeros = jnp.zeros((16,), jnp.float32)
    ones = zeros + 1.0

    def zstep(j, carry):
        for u in range(8):
            obuf[j * 8 + u, pl.ds(0, 16)] = zeros
        return carry

    lax.fori_loop(0, RW // 8, zstep, 0)

    def sstep(j, carry):
        for u in range(8):
            rows = lax.iota(jnp.int32, 16) + (j * 8 + u) * 16
            g = gbuf[pl.ds((j * 8 + u) * 16, 16)]
            plsc.store_scatter(obuf, [rows, g], ones)
        return carry

    lax.fori_loop(0, RW // 128, sstep, 0)
    pltpu.sync_copy(obuf, out_hbm.at[pl.ds(base, RW), :])


def _softmax_body(x_hbm, soft_hbm, ebuf, xbuf, sbuf, xsem, ssem):
    def xcopy(b, slot):
        return pltpu.make_async_copy(
            x_hbm.at[pl.ds(b * RB, RB), :], xbuf.at[slot], xsem.at[slot])

    def scopy(b, slot):
        return pltpu.make_async_copy(
            sbuf.at[slot], soft_hbm.at[pl.ds(b * RB, RB), :], ssem.at[slot])

    xcopy(0, 0).start()
    xcopy(1, 1).start()

    def pass1(b, acc):
        slot = jax.lax.rem(b, 2)
        xcopy(b, slot).wait()
        e = jnp.exp(xbuf[slot])
        ebuf[pl.ds(b * RB, RB), :] = e

        @pl.when(b + 2 < NB)
        def _prefetch():
            xcopy(b + 2, slot).start()

        return acc + jnp.sum(e)

    total = jax.lax.fori_loop(0, NB, pass1, 0.0)
    inv = 1.0 / total

    def pass2(b, carry):
        slot = jax.lax.rem(b, 2)

        @pl.when(b >= 2)
        def _drain():
            scopy(b - 2, slot).wait()

        sbuf[slot] = ebuf[pl.ds(b * RB, RB), :] * inv
        scopy(b, slot).start()
        return carry

    jax.lax.fori_loop(0, NB, pass2, 0)
    scopy(NB - 2, 0).wait()
    scopy(NB - 1, 1).wait()


def _softmax_tc(x):
    return pl.pallas_call(
        _softmax_body,
        in_specs=[pl.BlockSpec(memory_space=pltpu.MemorySpace.HBM)],
        out_specs=pl.BlockSpec(memory_space=pltpu.MemorySpace.HBM),
        out_shape=jax.ShapeDtypeStruct((B, C), jnp.float32),
        scratch_shapes=[
            pltpu.VMEM((B, C), jnp.float32),
            pltpu.VMEM((2, RB, C), jnp.float32),
            pltpu.VMEM((2, RB, C), jnp.float32),
            pltpu.SemaphoreType.DMA((2,)),
            pltpu.SemaphoreType.DMA((2,)),
        ],
    )(x)


def kernel(inference, ground_truth):
    ohe_wide = _ohe_sc(ground_truth.astype(jnp.int32))
    soft = _softmax_tc(inference)
    return (soft, ohe_wide[:, :C])


# speedup vs baseline: 1.3026x; 1.1097x over previous
import jax
import jax.numpy as jnp
from jax.experimental import pallas as pl
from jax.experimental.pallas import tpu as pltpu

B = 16384
C = 10
NB = 32
RB = B // NB
GB = 128 // (2 * NB)


def _body(x_hbm, g_hbm, soft_hbm, ohe_hbm,
          ebuf, xbuf, gbuf, obuf, sbuf, xsem, osem, ssem, gsem):
    def xcopy(b, slot):
        return pltpu.make_async_copy(
            x_hbm.at[pl.ds(b * RB, RB), :], xbuf.at[slot], xsem.at[slot])

    def ocopy(b, slot):
        return pltpu.make_async_copy(
            obuf.at[pl.ds(slot * GB, GB)], ohe_hbm.at[pl.ds(b * GB, GB)],
            osem.at[slot])

    def scopy(b, slot):
        return pltpu.make_async_copy(
            sbuf.at[slot], soft_hbm.at[pl.ds(b * RB, RB), :], ssem.at[slot])

    def ohe_block(b):
        slot = jax.lax.rem(b, 2)

        @pl.when(b >= 2)
        def _drain():
            ocopy(b - 2, slot).wait()

        g = gbuf[pl.ds(b * GB, GB), :]
        cls = jax.lax.broadcasted_iota(jnp.int32, (GB, 128, C), 2)
        obuf[pl.ds(slot * GB, GB)] = (g[:, :, None] == cls).astype(jnp.float32)
        ocopy(b, slot).start()

    gcopy = pltpu.make_async_copy(g_hbm, gbuf, gsem)
    gcopy.start()
    xcopy(0, 0).start()
    xcopy(1, 1).start()
    gcopy.wait()

    def pass1(b, acc):
        slot = jax.lax.rem(b, 2)
        xcopy(b, slot).wait()
        e = jnp.exp(xbuf[slot])
        ebuf[pl.ds(b * RB, RB), :] = e

        @pl.when(b + 2 < NB)
        def _prefetch():
            xcopy(b + 2, slot).start()

        ohe_block(b)
        return acc + jnp.sum(e)

    total = jax.lax.fori_loop(0, NB, pass1, 0.0)
    inv = 1.0 / total

    def pass2(b, carry):
        slot = jax.lax.rem(b, 2)

        @pl.when(b >= 2)
        def _drain():
            scopy(b - 2, slot).wait()

        sbuf[slot] = ebuf[pl.ds(b * RB, RB), :] * inv
        scopy(b, slot).start()
        ohe_block(NB + b)
        return carry

    jax.lax.fori_loop(0, NB, pass2, 0)

    ocopy(2 * NB - 2, 0).wait()
    ocopy(2 * NB - 1, 1).wait()
    scopy(NB - 2, 0).wait()
    scopy(NB - 1, 1).wait()


def kernel(inference, ground_truth):
    gt128 = ground_truth.astype(jnp.int32).reshape(128, 128)
    soft, ohe3 = pl.pallas_call(
        _body,
        in_specs=[
            pl.BlockSpec(memory_space=pltpu.MemorySpace.HBM),
            pl.BlockSpec(memory_space=pltpu.MemorySpace.HBM),
        ],
        out_specs=[
            pl.BlockSpec(memory_space=pltpu.MemorySpace.HBM),
            pl.BlockSpec(memory_space=pltpu.MemorySpace.HBM),
        ],
        out_shape=(
            jax.ShapeDtypeStruct((B, C), jnp.float32),
            jax.ShapeDtypeStruct((128, 128, C), jnp.float32),
        ),
        scratch_shapes=[
            pltpu.VMEM((B, C), jnp.float32),
            pltpu.VMEM((2, RB, C), jnp.float32),
            pltpu.VMEM((128, 128), jnp.int32),
            pltpu.VMEM((2 * GB, 128, C), jnp.float32),
            pltpu.VMEM((2, RB, C), jnp.float32),
            pltpu.SemaphoreType.DMA((2,)),
            pltpu.SemaphoreType.DMA((2,)),
            pltpu.SemaphoreType.DMA((2,)),
            pltpu.SemaphoreType.DMA,
        ],
    )(inference, gt128)
    return (soft, ohe3.reshape(B, C))


# speedup vs baseline: 2.0019x; 1.5369x over previous
import jax
import jax.numpy as jnp
from jax.experimental import pallas as pl
from jax.experimental.pallas import tpu as pltpu

B = 16384
C = 10
NB = 8
RB = B // NB
GB = 128 // NB


def _body(x_hbm, g_hbm, soft_hbm, ohe_hbm,
          ebuf, xbuf, gbuf, obuf, sbuf, xsem, osem, ssem, gsem):
    def xcopy(b, slot):
        return pltpu.make_async_copy(
            x_hbm.at[pl.ds(b * RB, RB), :], xbuf.at[slot], xsem.at[slot])

    def ocopy(b, slot):
        return pltpu.make_async_copy(
            obuf.at[slot], ohe_hbm.at[pl.ds(b * GB, GB)], osem.at[slot])

    def scopy(b, slot):
        return pltpu.make_async_copy(
            sbuf.at[slot], soft_hbm.at[pl.ds(b * RB, RB), :], ssem.at[slot])

    gcopy = pltpu.make_async_copy(g_hbm, gbuf, gsem)
    gcopy.start()
    xcopy(0, 0).start()
    xcopy(1, 1).start()
    gcopy.wait()

    def pass1(b, acc):
        slot = jax.lax.rem(b, 2)
        xcopy(b, slot).wait()
        e = jnp.exp(xbuf[slot])
        ebuf[pl.ds(b * RB, RB), :] = e

        @pl.when(b + 2 < NB)
        def _prefetch():
            xcopy(b + 2, slot).start()

        @pl.when(b >= 2)
        def _drain():
            ocopy(b - 2, slot).wait()

        g = gbuf[pl.ds(b * GB, GB), :]
        cls = jax.lax.broadcasted_iota(jnp.int32, (GB, 128, C), 2)
        obuf[slot] = (g[:, :, None] == cls).astype(jnp.float32)
        ocopy(b, slot).start()
        return acc + jnp.sum(e)

    total = jax.lax.fori_loop(0, NB, pass1, 0.0)
    inv = 1.0 / total

    def pass2(b, carry):
        slot = jax.lax.rem(b, 2)

        @pl.when(b >= 2)
        def _drain():
            scopy(b - 2, slot).wait()

        sbuf[slot] = ebuf[pl.ds(b * RB, RB), :] * inv
        scopy(b, slot).start()
        return carry

    jax.lax.fori_loop(0, NB, pass2, 0)

    ocopy(NB - 2, 0).wait()
    ocopy(NB - 1, 1).wait()
    scopy(NB - 2, 0).wait()
    scopy(NB - 1, 1).wait()


def kernel(inference, ground_truth):
    gt128 = ground_truth.astype(jnp.int32).reshape(128, 128)
    soft, ohe3 = pl.pallas_call(
        _body,
        in_specs=[
            pl.BlockSpec(memory_space=pltpu.MemorySpace.HBM),
            pl.BlockSpec(memory_space=pltpu.MemorySpace.HBM),
        ],
        out_specs=[
            pl.BlockSpec(memory_space=pltpu.MemorySpace.HBM),
            pl.BlockSpec(memory_space=pltpu.MemorySpace.HBM),
        ],
        out_shape=(
            jax.ShapeDtypeStruct((B, C), jnp.float32),
            jax.ShapeDtypeStruct((128, 128, C), jnp.float32),
        ),
        scratch_shapes=[
            pltpu.VMEM((B, C), jnp.float32),
            pltpu.VMEM((2, RB, C), jnp.float32),
            pltpu.VMEM((128, 128), jnp.int32),
            pltpu.VMEM((2, GB, 128, C), jnp.float32),
            pltpu.VMEM((2, RB, C), jnp.float32),
            pltpu.SemaphoreType.DMA((2,)),
            pltpu.SemaphoreType.DMA((2,)),
            pltpu.SemaphoreType.DMA((2,)),
            pltpu.SemaphoreType.DMA,
        ],
    )(inference, gt128)
    return (soft, ohe3.reshape(B, C))


# speedup vs baseline: 2.0986x; 1.0483x over previous
import jax
import jax.numpy as jnp
from jax.experimental import pallas as pl
from jax.experimental.pallas import tpu as pltpu

B = 16384
C = 10
NB = 4
RB = B // NB
GB = 128 // NB


def _body(x_hbm, g_hbm, soft_hbm, ohe_hbm,
          ebuf, xbuf, gbuf, obuf, sbuf, xsem, osem, ssem, gsem):
    def xcopy(b, slot):
        return pltpu.make_async_copy(
            x_hbm.at[pl.ds(b * RB, RB), :], xbuf.at[slot], xsem.at[slot])

    def ocopy(b, slot):
        return pltpu.make_async_copy(
            obuf.at[slot], ohe_hbm.at[pl.ds(b * GB, GB)], osem.at[slot])

    def scopy(b, slot):
        return pltpu.make_async_copy(
            sbuf.at[slot], soft_hbm.at[pl.ds(b * RB, RB), :], ssem.at[slot])

    gcopy = pltpu.make_async_copy(g_hbm, gbuf, gsem)
    gcopy.start()
    xcopy(0, 0).start()
    xcopy(1, 1).start()
    gcopy.wait()

    def pass1(b, acc):
        slot = jax.lax.rem(b, 2)
        xcopy(b, slot).wait()
        e = jnp.exp(xbuf[slot])
        ebuf[pl.ds(b * RB, RB), :] = e

        @pl.when(b + 2 < NB)
        def _prefetch():
            xcopy(b + 2, slot).start()

        @pl.when(b >= 2)
        def _drain():
            ocopy(b - 2, slot).wait()

        g = gbuf[pl.ds(b * GB, GB), :]
        cls = jax.lax.broadcasted_iota(jnp.int32, (GB, 128, C), 2)
        obuf[slot] = (g[:, :, None] == cls).astype(jnp.float32)
        ocopy(b, slot).start()
        return acc + jnp.sum(e)

    total = jax.lax.fori_loop(0, NB, pass1, 0.0)
    inv = 1.0 / total

    def pass2(b, carry):
        slot = jax.lax.rem(b, 2)

        @pl.when(b >= 2)
        def _drain():
            scopy(b - 2, slot).wait()

        sbuf[slot] = ebuf[pl.ds(b * RB, RB), :] * inv
        scopy(b, slot).start()
        return carry

    jax.lax.fori_loop(0, NB, pass2, 0)

    ocopy(NB - 2, 0).wait()
    ocopy(NB - 1, 1).wait()
    scopy(NB - 2, 0).wait()
    scopy(NB - 1, 1).wait()


def kernel(inference, ground_truth):
    gt128 = ground_truth.astype(jnp.int32).reshape(128, 128)
    soft, ohe3 = pl.pallas_call(
        _body,
        in_specs=[
            pl.BlockSpec(memory_space=pltpu.MemorySpace.HBM),
            pl.BlockSpec(memory_space=pltpu.MemorySpace.HBM),
        ],
        out_specs=[
            pl.BlockSpec(memory_space=pltpu.MemorySpace.HBM),
            pl.BlockSpec(memory_space=pltpu.MemorySpace.HBM),
        ],
        out_shape=(
            jax.ShapeDtypeStruct((B, C), jnp.float32),
            jax.ShapeDtypeStruct((128, 128, C), jnp.float32),
        ),
        scratch_shapes=[
            pltpu.VMEM((B, C), jnp.float32),
            pltpu.VMEM((2, RB, C), jnp.float32),
            pltpu.VMEM((128, 128), jnp.int32),
            pltpu.VMEM((2, GB, 128, C), jnp.float32),
            pltpu.VMEM((2, RB, C), jnp.float32),
            pltpu.SemaphoreType.DMA((2,)),
            pltpu.SemaphoreType.DMA((2,)),
            pltpu.SemaphoreType.DMA((2,)),
            pltpu.SemaphoreType.DMA,
        ],
    )(inference, gt128)
    return (soft, ohe3.reshape(B, C))


# speedup vs baseline: 2.1651x; 1.0317x over previous
import jax
import jax.numpy as jnp
from jax.experimental import pallas as pl
from jax.experimental.pallas import tpu as pltpu

B = 16384
C = 10
NB = 2
RB = B // NB
GB = 128 // NB


def _body(x_hbm, g_hbm, soft_hbm, ohe_hbm,
          ebuf, xbuf, gbuf, obuf, sbuf, xsem, osem, ssem, gsem):
    def xcopy(b, slot):
        return pltpu.make_async_copy(
            x_hbm.at[pl.ds(b * RB, RB), :], xbuf.at[slot], xsem.at[slot])

    def ocopy(b, slot):
        return pltpu.make_async_copy(
            obuf.at[slot], ohe_hbm.at[pl.ds(b * GB, GB)], osem.at[slot])

    def scopy(b, slot):
        return pltpu.make_async_copy(
            sbuf.at[slot], soft_hbm.at[pl.ds(b * RB, RB), :], ssem.at[slot])

    gcopy = pltpu.make_async_copy(g_hbm, gbuf, gsem)
    gcopy.start()
    xcopy(0, 0).start()
    xcopy(1, 1).start()
    gcopy.wait()

    def pass1(b, acc):
        slot = jax.lax.rem(b, 2)
        xcopy(b, slot).wait()
        e = jnp.exp(xbuf[slot])
        ebuf[pl.ds(b * RB, RB), :] = e

        @pl.when(b + 2 < NB)
        def _prefetch():
            xcopy(b + 2, slot).start()

        @pl.when(b >= 2)
        def _drain():
            ocopy(b - 2, slot).wait()

        g = gbuf[pl.ds(b * GB, GB), :]
        cls = jax.lax.broadcasted_iota(jnp.int32, (GB, 128, C), 2)
        obuf[slot] = (g[:, :, None] == cls).astype(jnp.float32)
        ocopy(b, slot).start()
        return acc + jnp.sum(e)

    total = jax.lax.fori_loop(0, NB, pass1, 0.0)
    inv = 1.0 / total

    def pass2(b, carry):
        slot = jax.lax.rem(b, 2)

        @pl.when(b >= 2)
        def _drain():
            scopy(b - 2, slot).wait()

        sbuf[slot] = ebuf[pl.ds(b * RB, RB), :] * inv
        scopy(b, slot).start()
        return carry

    jax.lax.fori_loop(0, NB, pass2, 0)

    ocopy(NB - 2, 0).wait()
    ocopy(NB - 1, 1).wait()
    scopy(NB - 2, 0).wait()
    scopy(NB - 1, 1).wait()


def kernel(inference, ground_truth):
    gt128 = ground_truth.astype(jnp.int32).reshape(128, 128)
    soft, ohe3 = pl.pallas_call(
        _body,
        in_specs=[
            pl.BlockSpec(memory_space=pltpu.MemorySpace.HBM),
            pl.BlockSpec(memory_space=pltpu.MemorySpace.HBM),
        ],
        out_specs=[
            pl.BlockSpec(memory_space=pltpu.MemorySpace.HBM),
            pl.BlockSpec(memory_space=pltpu.MemorySpace.HBM),
        ],
        out_shape=(
            jax.ShapeDtypeStruct((B, C), jnp.float32),
            jax.ShapeDtypeStruct((128, 128, C), jnp.float32),
        ),
        scratch_shapes=[
            pltpu.VMEM((B, C), jnp.float32),
            pltpu.VMEM((2, RB, C), jnp.float32),
            pltpu.VMEM((128, 128), jnp.int32),
            pltpu.VMEM((2, GB, 128, C), jnp.float32),
            pltpu.VMEM((2, RB, C), jnp.float32),
            pltpu.SemaphoreType.DMA((2,)),
            pltpu.SemaphoreType.DMA((2,)),
            pltpu.SemaphoreType.DMA((2,)),
            pltpu.SemaphoreType.DMA,
        ],
    )(inference, gt128)
    return (soft, ohe3.reshape(B, C))
